# baseline (speedup 1.0000x reference)
import functools
import math

import jax
import jax.numpy as jnp
from jax import lax
from jax.experimental import pallas as pl
from jax.experimental.pallas import tpu as pltpu
from jax.experimental.pallas import tpu_sc as plsc

N_T = 1000
N_PAD = 1024
OSC = 112
HID = 192
N_STEPS = 5
DT = 0.01
THRESH = 0.1
TWO_PI = 2.0 * math.pi
EPS = 1e-8
NEG = -3.0e38
BIG = 1 << 30
LANES = 16
PRIV = LANES * N_PAD
NCHUNK = N_PAD // LANES


def _tc_body(dt_hbm, dt1_hbm, wp1_hbm, bp1_hbm, wp2_hbm, bp2_hbm, wp3_hbm,
             bp3_hbm, wr1_hbm, br1_hbm, wr2_hbm, br2_hbm, om_hbm,
             sim_ref, msim_ref, midx_ref,
             dt_ref, dt1_ref, wp1_ref, bp1_ref, wp2_ref, bp2_ref, wp3_ref,
             bp3_ref, wr1_ref, br1_ref, wr2_ref, br2_ref, om_ref, sem):
    f32 = jnp.float32
    cn = (((0,), (0,)), ((), ()))
    sn = (((1,), (0,)), ((), ()))

    copies = [pltpu.make_async_copy(h, v, sem) for h, v in (
        (dt_hbm, dt_ref), (dt1_hbm, dt1_ref), (wp1_hbm, wp1_ref),
        (bp1_hbm, bp1_ref), (wp2_hbm, wp2_ref), (bp2_hbm, bp2_ref),
        (wp3_hbm, wp3_ref), (bp3_hbm, bp3_ref), (wr1_hbm, wr1_ref),
        (br1_hbm, br1_ref), (wr2_hbm, wr2_ref), (br2_hbm, br2_ref),
        (om_hbm, om_ref))]
    for c in copies:
        c.start()
    for c in copies:
        c.wait()

    bp1c = bp1_ref[...][:, None]
    bp2c = bp2_ref[...][:, None]
    bp3c = bp3_ref[...][:, None]
    br1c = br1_ref[...][:, None]
    br2c = br2_ref[...][:, None]

    def mlp_phase(dT):
        h = jnp.maximum(
            lax.dot_general(wp1_ref[...], dT, cn, preferred_element_type=f32)
            + bp1c, 0.0)
        h = jnp.maximum(
            lax.dot_general(wp2_ref[...], h, sn, preferred_element_type=f32)
            + bp2c, 0.0)
        return lax.dot_general(wp3_ref[...], h, cn,
                               preferred_element_type=f32) + bp3c

    ph_t = jnp.mod(mlp_phase(dt_ref[...]), TWO_PI)
    ph_1 = jnp.mod(mlp_phase(dt1_ref[...]), TWO_PI)

    step = om_ref[...][:, None] * f32(TWO_PI * DT)
    ph = ph_t
    for _ in range(N_STEPS):
        ph = ph + step
    ph = jnp.mod(ph, TWO_PI)
    hr = jnp.maximum(
        lax.dot_general(wr1_ref[...], ph, sn, preferred_element_type=f32)
        + br1c, 0.0)
    r = lax.dot_general(wr2_ref[...], hr, cn, preferred_element_type=f32) + br2c
    ph = jnp.mod(ph + 0.1 * r, TWO_PI)

    ca, sa = jnp.cos(ph), jnp.sin(ph)
    cb, sb = jnp.cos(ph_1), jnp.sin(ph_1)
    na = jnp.sqrt(jnp.sum(ca * ca + sa * sa, axis=0, keepdims=True)) + EPS
    nb = jnp.sqrt(jnp.sum(cb * cb + sb * sb, axis=0, keepdims=True)) + EPS
    sim = (lax.dot_general(ca / na, cb / nb, cn, preferred_element_type=f32)
           + lax.dot_general(sa / na, sb / nb, cn, preferred_element_type=f32))

    sim_ref[...] = sim

    colf = lax.broadcasted_iota(jnp.int32, (1, N_T), 1).astype(f32)
    msim = jnp.max(sim, axis=1, keepdims=True)
    midxf = jnp.min(jnp.where(sim == msim, colf, f32(3e38)), axis=1,
                    keepdims=True)
    midx = midxf.astype(jnp.int32)

    msim_ref[0:N_T] = msim[:, 0]
    msim_ref[N_T:N_PAD] = jnp.full((N_PAD - N_T,), NEG, f32)
    midx_ref[0:N_T] = midx[:, 0]
    midx_ref[N_T:N_PAD] = jnp.zeros((N_PAD - N_T,), jnp.int32)


_tc_call = pl.pallas_call(
    _tc_body,
    in_specs=[pl.BlockSpec(memory_space=pl.ANY)] * 13,
    out_shape=(
        jax.ShapeDtypeStruct((N_T, N_T), jnp.float32),
        jax.ShapeDtypeStruct((N_PAD,), jnp.float32),
        jax.ShapeDtypeStruct((N_PAD,), jnp.int32),
    ),
    scratch_shapes=[
        pltpu.VMEM((4, N_T), jnp.float32),
        pltpu.VMEM((4, N_T), jnp.float32),
        pltpu.VMEM((4, HID), jnp.float32),
        pltpu.VMEM((HID,), jnp.float32),
        pltpu.VMEM((HID, HID), jnp.float32),
        pltpu.VMEM((HID,), jnp.float32),
        pltpu.VMEM((HID, OSC), jnp.float32),
        pltpu.VMEM((OSC,), jnp.float32),
        pltpu.VMEM((HID, OSC), jnp.float32),
        pltpu.VMEM((HID,), jnp.float32),
        pltpu.VMEM((HID, OSC), jnp.float32),
        pltpu.VMEM((OSC,), jnp.float32),
        pltpu.VMEM((OSC,), jnp.float32),
        pltpu.SemaphoreType.DMA,
    ],
)


def _sc_body(msim_hbm, midx_hbm, initf_hbm, initi_hbm, out_hbm,
             sims_v, idxs_v, priv_f, best_f, priv_i, best_i, match_v, sem):
    is0 = jnp.logical_and(lax.axis_index("c") == 0, lax.axis_index("s") == 0)
    UNR = 4

    @pl.when(is0)
    def _():
        c1 = pltpu.make_async_copy(msim_hbm, sims_v, sem)
        c2 = pltpu.make_async_copy(midx_hbm, idxs_v, sem)
        c3 = pltpu.make_async_copy(initf_hbm, priv_f, sem)
        c4 = pltpu.make_async_copy(initi_hbm, priv_i, sem)
        c1.start(); c2.start(); c3.start(); c4.start()
        c1.wait(); c2.wait(); c3.wait(); c4.wait()

        lanes = lax.iota(jnp.int32, 16)
        laneoff = lanes * N_PAD

        def pass_a(kk, carry):
            for u in range(UNR):
                off = pl.multiple_of(kk * (LANES * UNR) + u * LANES, LANES)
                v = sims_v[pl.ds(off, LANES)]
                ix = idxs_v[pl.ds(off, LANES)]
                addr = laneoff + ix
                cur = plsc.load_gather(priv_f, [addr])
                plsc.store_scatter(priv_f, [addr], jnp.maximum(cur, v))
            return carry
        lax.fori_loop(0, NCHUNK // UNR, pass_a, 0)

        def red_a(kk, carry):
            for u in range(2):
                off = pl.multiple_of(kk * (LANES * 2) + u * LANES, LANES)
                acc = priv_f[pl.ds(off, LANES)]
                for l in range(1, LANES):
                    acc = jnp.maximum(acc, priv_f[pl.ds(off + l * N_PAD, LANES)])
                best_f[pl.ds(off, LANES)] = acc
            return carry
        lax.fori_loop(0, NCHUNK // 2, red_a, 0)

        def pass_b(kk, carry):
            for u in range(UNR):
                k = kk * UNR + u
                off = pl.multiple_of(k * LANES, LANES)
                v = sims_v[pl.ds(off, LANES)]
                ix = idxs_v[pl.ds(off, LANES)]
                bj = plsc.load_gather(best_f, [ix])
                cand = jnp.logical_and(v == bj, v >= THRESH)
                rows = k * LANES + lanes
                addr = laneoff + ix
                cur = plsc.load_gather(priv_i, [addr])
                plsc.store_scatter(priv_i, [addr],
                                   jnp.where(cand, jnp.minimum(cur, rows), cur))
            return carry
        lax.fori_loop(0, NCHUNK // UNR, pass_b, 0)

        def red_b(kk, carry):
            for u in range(2):
                off = pl.multiple_of(kk * (LANES * 2) + u * LANES, LANES)
                acc = priv_i[pl.ds(off, LANES)]
                for l in range(1, LANES):
                    acc = jnp.minimum(acc, priv_i[pl.ds(off + l * N_PAD, LANES)])
                best_i[pl.ds(off, LANES)] = acc
            return carry
        lax.fori_loop(0, NCHUNK // 2, red_b, 0)

        def pass_c(kk, carry):
            for u in range(UNR):
                k = kk * UNR + u
                off = pl.multiple_of(k * LANES, LANES)
                ix = idxs_v[pl.ds(off, LANES)]
                w = plsc.load_gather(best_i, [ix])
                rows = k * LANES + lanes
                match_v[pl.ds(off, LANES)] = jnp.where(w == rows, ix, -1)
            return carry
        lax.fori_loop(0, NCHUNK // UNR, pass_c, 0)

        pltpu.sync_copy(match_v, out_hbm)


def _make_sc_match():
    return functools.partial(
        pl.kernel,
        mesh=plsc.VectorSubcoreMesh(core_axis_name="c", subcore_axis_name="s"),
        out_type=jax.ShapeDtypeStruct((N_PAD,), jnp.int32),
        compiler_params=pltpu.CompilerParams(needs_layout_passes=False),
        scratch_types=[
            pltpu.VMEM((N_PAD,), jnp.float32),
            pltpu.VMEM((N_PAD,), jnp.int32),
            pltpu.VMEM((PRIV,), jnp.float32),
            pltpu.VMEM((N_PAD,), jnp.float32),
            pltpu.VMEM((PRIV,), jnp.int32),
            pltpu.VMEM((N_PAD,), jnp.int32),
            pltpu.VMEM((N_PAD,), jnp.int32),
            pltpu.SemaphoreType.DMA,
        ],
    )(_sc_body)


def kernel(detections_t, detections_t1, Wp1, bp1, Wp2, bp2, Wp3, bp3,
           Wa1, ba1, Wa2, ba2, Wr1, br1, Wr2, br2, omega):
    f32 = jnp.float32
    sim, msim, midx = _tc_call(detections_t.T, detections_t1.T, Wp1.T, bp1,
                               Wp2, bp2, Wp3.T, bp3, Wr1, br1, Wr2.T, br2,
                               omega)

    initf = jnp.full((PRIV,), NEG, f32)
    initi = jnp.full((PRIV,), BIG, jnp.int32)
    matches = _make_sc_match()(msim, midx, initf, initi)
    return matches[:N_T], sim
